# Initial kernel scaffold; baseline (speedup 1.0000x reference)
import jax
import jax.numpy as jnp
from jax.experimental import pallas as pl


def kernel(x, codebooks):
    raise NotImplementedError("write your pallas kernel here")


# speedup vs baseline: 1.3384x; 1.3384x over previous
import functools

import jax
import jax.numpy as jnp
from jax.experimental import pallas as pl
from jax.experimental.pallas import tpu as pltpu

B, D, T = 8, 128, 4096
NQ, CS = 30, 1024
N = B * T
TILE = 512
NT = N // TILE


def _rvq_kernel(x_ref, cb_ref, out_ref, loss_ref, res_ref):
    j = pl.program_id(1)

    @pl.when(j == 0)
    def _init():
        res_ref[...] = x_ref[...]
        loss_ref[...] = jnp.zeros_like(loss_ref)

    res = res_ref[...]
    cb = cb_ref[0]

    mm = jax.lax.dot_general(res.astype(jnp.bfloat16), cb.astype(jnp.bfloat16),
                             (((1,), (1,)), ((), ())),
                             preferred_element_type=jnp.float32)
    sx = jnp.sum(res * res, axis=1, keepdims=True)
    sc = jnp.sum(cb * cb, axis=1)[None, :]
    d2 = (sx - 2.0 * mm) + sc

    iota = jax.lax.broadcasted_iota(jnp.int32, (TILE, CS), 1)
    dmin = jnp.min(d2, axis=1, keepdims=True)
    idx = jnp.min(jnp.where(d2 == dmin, iota, CS), axis=1)
    oh = (iota == idx[:, None]).astype(jnp.bfloat16)
    hi = cb.astype(jnp.bfloat16)
    r1 = cb - hi.astype(jnp.float32)
    lo1 = r1.astype(jnp.bfloat16)
    lo2 = (r1 - lo1.astype(jnp.float32)).astype(jnp.bfloat16)
    dn = (((1,), (0,)), ((), ()))
    q_hi = jax.lax.dot_general(oh, hi, dn, preferred_element_type=jnp.float32)
    q_l1 = jax.lax.dot_general(oh, lo1, dn, preferred_element_type=jnp.float32)
    q_l2 = jax.lax.dot_general(oh, lo2, dn, preferred_element_type=jnp.float32)
    q = (q_hi + q_l1) + q_l2

    loss_ref[...] += jnp.sum((q - res) * (q - res), axis=0)[None, None, :]
    res_ref[...] = res - q

    @pl.when(j == NQ - 1)
    def _fin():
        out_ref[...] = x_ref[...] - res_ref[...]


@jax.jit
def kernel(x, codebooks):
    xr = jnp.transpose(x, (0, 2, 1)).reshape(N, D)
    out, loss = pl.pallas_call(
        _rvq_kernel,
        grid=(NT, NQ),
        in_specs=[
            pl.BlockSpec((TILE, D), lambda i, j: (i, 0)),
            pl.BlockSpec((1, CS, D), lambda i, j: (j, 0, 0)),
        ],
        out_specs=[
            pl.BlockSpec((TILE, D), lambda i, j: (i, 0)),
            pl.BlockSpec((1, 1, D), lambda i, j: (i, 0, 0)),
        ],
        out_shape=[
            jax.ShapeDtypeStruct((N, D), jnp.float32),
            jax.ShapeDtypeStruct((NT, 1, D), jnp.float32),
        ],
        scratch_shapes=[pltpu.VMEM((TILE, D), jnp.float32)],
    )(xr, codebooks)
    quantized = jnp.transpose(out.reshape(B, T, D), (0, 2, 1))
    commit_loss = jnp.sum(loss) / jnp.float32(N * D)
    return quantized, commit_loss


# speedup vs baseline: 1.6098x; 1.2028x over previous
import functools

import jax
import jax.numpy as jnp
from jax.experimental import pallas as pl
from jax.experimental.pallas import tpu as pltpu

B, D, T = 8, 128, 4096
NQ, CS = 30, 1024
N = B * T
TILE = 2048
NT = N // TILE


def _rvq_kernel(x_ref, cb_ref, out_ref, loss_ref, res_ref):
    j = pl.program_id(1)

    @pl.when(j == 0)
    def _init():
        res_ref[...] = x_ref[...]
        loss_ref[...] = jnp.zeros_like(loss_ref)

    res = res_ref[...]
    cb = cb_ref[0]

    mm2 = jax.lax.dot_general(res.astype(jnp.bfloat16) * jnp.bfloat16(-2.0),
                              cb.astype(jnp.bfloat16),
                              (((1,), (1,)), ((), ())),
                              preferred_element_type=jnp.float32)
    sx = jnp.sum(res * res, axis=1, keepdims=True)
    sc = jnp.sum(cb * cb, axis=1)[None, :]
    d2 = (sx + mm2) + sc

    iota = jax.lax.broadcasted_iota(jnp.int32, (TILE, CS), 1)
    dmin = jnp.min(d2, axis=1, keepdims=True)
    idx = jnp.min(jnp.where(d2 == dmin, iota, CS), axis=1)
    oh = (iota == idx[:, None]).astype(jnp.bfloat16)
    hi = cb.astype(jnp.bfloat16)
    r1 = cb - hi.astype(jnp.float32)
    lo1 = r1.astype(jnp.bfloat16)
    lo2 = (r1 - lo1.astype(jnp.float32)).astype(jnp.bfloat16)
    dn = (((1,), (0,)), ((), ()))
    q_hi = jax.lax.dot_general(oh, hi, dn, preferred_element_type=jnp.float32)
    q_l1 = jax.lax.dot_general(oh, lo1, dn, preferred_element_type=jnp.float32)
    q_l2 = jax.lax.dot_general(oh, lo2, dn, preferred_element_type=jnp.float32)
    q = (q_hi + q_l1) + q_l2

    loss_ref[...] += jnp.sum((q - res) * (q - res), axis=0)[None, None, :]
    res_ref[...] = res - q

    @pl.when(j == NQ - 1)
    def _fin():
        out_ref[...] = x_ref[...] - res_ref[...]


@jax.jit
def kernel(x, codebooks):
    xr = jnp.transpose(x, (0, 2, 1)).reshape(N, D)
    out, loss = pl.pallas_call(
        _rvq_kernel,
        grid=(NT, NQ),
        in_specs=[
            pl.BlockSpec((TILE, D), lambda i, j: (i, 0)),
            pl.BlockSpec((1, CS, D), lambda i, j: (j, 0, 0)),
        ],
        out_specs=[
            pl.BlockSpec((TILE, D), lambda i, j: (i, 0)),
            pl.BlockSpec((1, 1, D), lambda i, j: (i, 0, 0)),
        ],
        out_shape=[
            jax.ShapeDtypeStruct((N, D), jnp.float32),
            jax.ShapeDtypeStruct((NT, 1, D), jnp.float32),
        ],
        scratch_shapes=[pltpu.VMEM((TILE, D), jnp.float32)],
    )(xr, codebooks)
    quantized = jnp.transpose(out.reshape(B, T, D), (0, 2, 1))
    commit_loss = jnp.sum(loss) / jnp.float32(N * D)
    return quantized, commit_loss
